# Initial kernel scaffold; baseline (speedup 1.0000x reference)
import functools

import jax
import jax.numpy as jnp
from jax import lax
from jax.experimental import pallas as pl
from jax.experimental.pallas import tpu as pltpu
from jax.experimental.pallas import tpu_sc as plsc

NC = 2
NS = 16
L = 16
NW = NC * NS


def _make_kernel(B, S, D):
    n_rows = B * S
    assert n_rows % NW == 0
    rows_per_w = n_rows // NW
    assert rows_per_w % S == 0
    seqs_per_w = rows_per_w // S
    g0 = min(128, S)
    g1 = S - g0

    mesh = plsc.VectorSubcoreMesh(
        core_axis_name="c", subcore_axis_name="s", num_cores=NC,
        num_subcores=NS)

    @functools.partial(
        pl.kernel,
        out_type=jax.ShapeDtypeStruct((n_rows, D), jnp.float32),
        mesh=mesh,
        scratch_types=[
            pltpu.VMEM((rows_per_w,), jnp.int32),
            pltpu.VMEM((S, D), jnp.float32),
            pltpu.VMEM((S, D), jnp.float32),
            pltpu.SemaphoreType.DMA,
        ],
    )
    def emb_kernel(emb_hbm, idx_hbm, pos_hbm, out_hbm, idx_v, pos_v,
                   rows_v, sem):
        wid = lax.axis_index("s") * NC + lax.axis_index("c")
        base = wid * rows_per_w

        pltpu.sync_copy(idx_hbm.at[pl.ds(base, rows_per_w)], idx_v)
        pltpu.sync_copy(pos_hbm.at[pl.ds(0, S)], pos_v)

        def per_seq(q, carry):
            row0 = q * S
            cp0 = pltpu.async_copy(
                emb_hbm.at[idx_v.at[pl.ds(row0, g0)]],
                rows_v.at[pl.ds(0, g0)], sem)
            cp1 = pltpu.async_copy(
                emb_hbm.at[idx_v.at[pl.ds(row0 + g0, g1)]],
                rows_v.at[pl.ds(g0, g1)], sem)
            cp0.wait()
            cp1.wait()

            def add_pos(s, carry2):
                for j in range(D // L):
                    sl = pl.ds(j * L, L)
                    rows_v[s, sl] = rows_v[s, sl] + pos_v[s, sl]
                return carry2

            lax.fori_loop(0, S, add_pos, 0)
            pltpu.sync_copy(rows_v, out_hbm.at[pl.ds(base + row0, S)])
            return carry

        lax.fori_loop(0, seqs_per_w, per_seq, 0)

    return emb_kernel


def kernel(x, emb_table, pos_table):
    B, S = x.shape
    D = emb_table.shape[1]
    xf = x.reshape(B * S).astype(jnp.int32)
    out = _make_kernel(B, S, D)(emb_table, xf, pos_table)
    return out.reshape(B, S, D)


# speedup vs baseline: 2.7169x; 2.7169x over previous
import functools

import jax
import jax.numpy as jnp
from jax import lax
from jax.experimental import pallas as pl
from jax.experimental.pallas import tpu as pltpu
from jax.experimental.pallas import tpu_sc as plsc

NC = 2
NS = 16
L = 16
NW = NC * NS


def _make_kernel(B, S, D):
    n_rows = B * S
    assert n_rows % NW == 0
    rows_per_w = n_rows // NW
    assert rows_per_w % S == 0
    seqs_per_w = rows_per_w // S
    g0 = min(128, S)
    g1 = S - g0

    mesh = plsc.VectorSubcoreMesh(
        core_axis_name="c", subcore_axis_name="s", num_cores=NC,
        num_subcores=NS)

    @functools.partial(
        pl.kernel,
        out_type=jax.ShapeDtypeStruct((n_rows, D), jnp.float32),
        mesh=mesh,
        scratch_types=[
            pltpu.VMEM((rows_per_w,), jnp.int32),
            pltpu.VMEM((S, D), jnp.float32),
            pltpu.VMEM((S, D), jnp.float32),
            pltpu.SemaphoreType.DMA,
        ],
        compiler_params=pltpu.CompilerParams(use_tc_tiling_on_sc=False),
    )
    def emb_kernel(emb_hbm, idx_hbm, pos_hbm, out_hbm, idx_v, pos_v,
                   rows_v, sem):
        wid = lax.axis_index("s") * NC + lax.axis_index("c")
        base = wid * rows_per_w

        pltpu.sync_copy(idx_hbm.at[pl.ds(base, rows_per_w)], idx_v)
        pltpu.sync_copy(pos_hbm.at[pl.ds(0, S)], pos_v)

        def per_seq(q, carry):
            row0 = q * S
            cp0 = pltpu.async_copy(
                emb_hbm.at[idx_v.at[pl.ds(row0, g0)]],
                rows_v.at[pl.ds(0, g0)], sem)
            cp1 = pltpu.async_copy(
                emb_hbm.at[idx_v.at[pl.ds(row0 + g0, g1)]],
                rows_v.at[pl.ds(g0, g1)], sem)
            cp0.wait()
            cp1.wait()

            def add_pos(s, carry2):
                for j in range(D // L):
                    sl = pl.ds(j * L, L)
                    rows_v[s, sl] = rows_v[s, sl] + pos_v[s, sl]
                return carry2

            lax.fori_loop(0, S, add_pos, 0)
            pltpu.sync_copy(rows_v, out_hbm.at[pl.ds(base + row0, S)])
            return carry

        lax.fori_loop(0, seqs_per_w, per_seq, 0)

    return emb_kernel


def kernel(x, emb_table, pos_table):
    B, S = x.shape
    D = emb_table.shape[1]
    xf = x.reshape(B * S).astype(jnp.int32)
    out = _make_kernel(B, S, D)(emb_table, xf, pos_table)
    return out.reshape(B, S, D)


# speedup vs baseline: 2.8956x; 1.0658x over previous
import functools

import jax
import jax.numpy as jnp
from jax import lax
from jax.experimental import pallas as pl
from jax.experimental.pallas import tpu as pltpu
from jax.experimental.pallas import tpu_sc as plsc

NC = 2
NS = 16
L = 16
NW = NC * NS
NBUF = 6
LOOKAHEAD = 3


def _make_kernel(B, S, D):
    n_rows = B * S
    assert n_rows % NW == 0
    rows_per_w = n_rows // NW
    assert rows_per_w % S == 0
    n_seq = rows_per_w // S
    assert n_seq >= NBUF
    g0 = min(128, S)
    g1 = S - g0

    mesh = plsc.VectorSubcoreMesh(
        core_axis_name="c", subcore_axis_name="s", num_cores=NC,
        num_subcores=NS)

    @functools.partial(
        pl.kernel,
        out_type=jax.ShapeDtypeStruct((n_rows, D), jnp.float32),
        mesh=mesh,
        scratch_types=[
            pltpu.VMEM((rows_per_w,), jnp.int32),
            pltpu.VMEM((S, D), jnp.float32),
            pltpu.VMEM((NBUF, S, D), jnp.float32),
            pltpu.SemaphoreType.DMA((NBUF,)),
            pltpu.SemaphoreType.DMA((NBUF,)),
        ],
        compiler_params=pltpu.CompilerParams(use_tc_tiling_on_sc=False),
    )
    def emb_kernel(emb_hbm, idx_hbm, pos_hbm, out_hbm, idx_v, pos_v,
                   rows_v, g_sem, w_sem):
        wid = lax.axis_index("s") * NC + lax.axis_index("c")
        base = wid * rows_per_w

        pltpu.sync_copy(idx_hbm.at[pl.ds(base, rows_per_w)], idx_v)
        pltpu.sync_copy(pos_hbm.at[pl.ds(0, S)], pos_v)

        def start_gather(q, b):
            row0 = q * S
            pltpu.async_copy(
                emb_hbm.at[idx_v.at[pl.ds(row0, g0)]],
                rows_v.at[b, pl.ds(0, g0)], g_sem.at[b])
            pltpu.async_copy(
                emb_hbm.at[idx_v.at[pl.ds(row0 + g0, g1)]],
                rows_v.at[b, pl.ds(g0, g1)], g_sem.at[b])

        def wait_gather(b):
            pltpu.make_async_copy(
                emb_hbm.at[idx_v.at[pl.ds(0, g0)]],
                rows_v.at[b, pl.ds(0, g0)], g_sem.at[b]).wait()
            pltpu.make_async_copy(
                emb_hbm.at[idx_v.at[pl.ds(g0, g1)]],
                rows_v.at[b, pl.ds(g0, g1)], g_sem.at[b]).wait()

        def start_write(q, b):
            pltpu.async_copy(
                rows_v.at[b], out_hbm.at[pl.ds(base + q * S, S)],
                w_sem.at[b])

        def wait_write(b):
            pltpu.make_async_copy(
                rows_v.at[b], out_hbm.at[pl.ds(base, S)],
                w_sem.at[b]).wait()

        def add_pos(b):
            def body(s4, carry):
                for i in range(4):
                    for j in range(D // L):
                        sl = pl.ds(j * L, L)
                        s = s4 * 4 + i
                        rows_v[b, s, sl] = rows_v[b, s, sl] + pos_v[s, sl]
                return carry
            lax.fori_loop(0, S // 4, body, 0, unroll=2)

        for q in range(LOOKAHEAD):
            start_gather(q, q)

        def slot(q, b):
            qn = q + LOOKAHEAD
            bn = (b + LOOKAHEAD) % NBUF

            @pl.when(jnp.logical_and(qn < n_seq, qn >= NBUF))
            def _():
                wait_write(bn)

            @pl.when(qn < n_seq)
            def _():
                start_gather(qn, bn)

            wait_gather(b)
            add_pos(b)
            start_write(q, b)

        def group(k, carry):
            for b in range(NBUF):
                slot(k * NBUF + b, b)
            return carry

        n_groups = n_seq // NBUF
        lax.fori_loop(0, n_groups, group, 0)
        for q in range(n_groups * NBUF, n_seq):
            slot(q, q % NBUF)

        for q in range(n_seq - NBUF, n_seq):
            wait_write(q % NBUF)

    return emb_kernel


def kernel(x, emb_table, pos_table):
    B, S = x.shape
    D = emb_table.shape[1]
    xf = x.reshape(B * S).astype(jnp.int32)
    out = _make_kernel(B, S, D)(emb_table, xf, pos_table)
    return out.reshape(B, S, D)
